# Initial kernel scaffold; baseline (speedup 1.0000x reference)
import jax
import jax.numpy as jnp
from jax.experimental import pallas as pl


def kernel(x, edge_index, bit_sum, W, bias, bn_gamma, bn_beta):
    raise NotImplementedError("write your pallas kernel here")


# speedup vs baseline: 17.6731x; 17.6731x over previous
import functools

import jax
import jax.numpy as jnp
from jax import lax
from jax.experimental import pallas as pl
from jax.experimental.pallas import tpu as pltpu
from jax.experimental.pallas import tpu_sc as plsc

_N = 10000
_E = 320000
_D = 128
_NC = 2
_NS = 16
_NW = _NC * _NS
_NPAD = 10240
_RPT = _NPAD // _NS
_EPW = _E // _NW
_CH = 128
_NCH = _EPW // _CH
_TAIL = _EPW - _NCH * _CH

_MESH = dict(core_axis_name="c", subcore_axis_name="s")


@functools.partial(
    pl.kernel,
    out_type=jax.ShapeDtypeStruct((_NC * _NPAD,), jnp.float32),
    mesh=plsc.VectorSubcoreMesh(**_MESH),
    scratch_types=[
        pltpu.VMEM((_CH,), jnp.int32),
        pltpu.VMEM((_TAIL,), jnp.int32),
        pltpu.VMEM((_CH,), jnp.float32),
        pltpu.VMEM((_TAIL,), jnp.float32),
        pltpu.VMEM((_RPT,), jnp.float32),
        pltpu.VMEM_SHARED((_NPAD,), jnp.float32),
    ],
)
def _deg_call(col_hbm, deg_hbm, coli, colt, ones_c, ones_t, zbuf, deg_sh):
    cid = lax.axis_index("c")
    sid = lax.axis_index("s")
    w = cid * _NS + sid
    zeros16 = jnp.zeros((16,), jnp.float32)
    ones16 = jnp.ones((16,), jnp.float32)
    for j in range(_CH // 16):
        ones_c[pl.ds(j * 16, 16)] = ones16
    ones_t[...] = ones16
    for j in range(_RPT // 16):
        zbuf[pl.ds(j * 16, 16)] = zeros16
    pltpu.sync_copy(zbuf, deg_sh.at[pl.ds(sid * _RPT, _RPT)])
    plsc.subcore_barrier()
    base0 = pl.multiple_of(w * _EPW, 8)

    def body(i, carry):
        b = pl.multiple_of(base0 + i * _CH, 8)
        pltpu.sync_copy(col_hbm.at[pl.ds(b, _CH)], coli)
        pltpu.sync_copy(ones_c, deg_sh.at[coli], add=True)
        return carry

    lax.fori_loop(0, _NCH, body, 0)
    bt = pl.multiple_of(base0 + _NCH * _CH, 8)
    pltpu.sync_copy(col_hbm.at[pl.ds(bt, _TAIL)], colt)
    pltpu.sync_copy(ones_t, deg_sh.at[colt], add=True)
    plsc.subcore_barrier()
    pltpu.sync_copy(
        deg_sh.at[pl.ds(sid * _RPT, _RPT)],
        deg_hbm.at[pl.ds(cid * _NPAD + sid * _RPT, _RPT)],
    )


@functools.partial(
    pl.kernel,
    out_type=jax.ShapeDtypeStruct((_NC * _NPAD, _D), jnp.float32),
    mesh=plsc.VectorSubcoreMesh(**_MESH),
    scratch_types=[
        pltpu.VMEM((_CH,), jnp.int32),
        pltpu.VMEM((_CH,), jnp.int32),
        pltpu.VMEM((_TAIL,), jnp.int32),
        pltpu.VMEM((_TAIL,), jnp.int32),
        pltpu.VMEM((_CH, _D), jnp.float32),
        pltpu.VMEM((_TAIL, _D), jnp.float32),
        pltpu.VMEM((16, _D), jnp.float32),
        pltpu.VMEM_SHARED((_NPAD, _D), jnp.float32),
        pltpu.SemaphoreType.DMA,
    ],
)
def _msg_call(row_hbm, col_hbm, g_hbm, out_hbm,
              rowi, coli, rowt, colt, rows, rowst, zb, acc_sh, sem):
    cid = lax.axis_index("c")
    sid = lax.axis_index("s")
    w = cid * _NS + sid
    zeros16 = jnp.zeros((16,), jnp.float32)
    for i in range(16):
        for j in range(_D // 16):
            zb[i, pl.ds(j * 16, 16)] = zeros16
    for t in range(_RPT // 16):
        pltpu.sync_copy(zb, acc_sh.at[pl.ds(sid * _RPT + t * 16, 16)])
    plsc.subcore_barrier()
    base0 = pl.multiple_of(w * _EPW, 8)

    def body(i, carry):
        b = pl.multiple_of(base0 + i * _CH, 8)
        pltpu.sync_copy(row_hbm.at[pl.ds(b, _CH)], rowi)
        pltpu.sync_copy(col_hbm.at[pl.ds(b, _CH)], coli)
        pltpu.async_copy(g_hbm.at[rowi], rows, sem).wait()
        pltpu.sync_copy(rows, acc_sh.at[coli], add=True)
        return carry

    lax.fori_loop(0, _NCH, body, 0)
    bt = pl.multiple_of(base0 + _NCH * _CH, 8)
    pltpu.sync_copy(row_hbm.at[pl.ds(bt, _TAIL)], rowt)
    pltpu.sync_copy(col_hbm.at[pl.ds(bt, _TAIL)], colt)
    pltpu.async_copy(g_hbm.at[rowt], rowst, sem).wait()
    pltpu.sync_copy(rowst, acc_sh.at[colt], add=True)
    plsc.subcore_barrier()
    pltpu.sync_copy(
        acc_sh.at[pl.ds(sid * _RPT, _RPT)],
        out_hbm.at[pl.ds(cid * _NPAD + sid * _RPT, _RPT)],
    )


def _lin_body(x_ref, w_ref, degp_ref, g_ref, dis_ref):
    deg = degp_ref[0, :] + degp_ref[1, :]
    dis = jnp.where(deg > 0.0, lax.rsqrt(deg), 0.0)
    dis_ref[...] = dis
    h = lax.dot_general(
        x_ref[...], w_ref[...], (((1,), (1,)), ((), ())),
        preferred_element_type=jnp.float32,
    )
    g_ref[...] = h * dis[:_N][:, None]


def _bn_body(s_ref, dis_ref, b_ref, g_ref, be_ref, x_ref, o_ref):
    s = s_ref[pl.ds(0, _N), :] + s_ref[pl.ds(_NPAD, _N), :]
    pre = s * dis_ref[pl.ds(0, _N)][:, None] + b_ref[...][None, :]
    mean = jnp.mean(pre, axis=0)
    cen = pre - mean[None, :]
    var = jnp.mean(cen * cen, axis=0)
    o_ref[...] = (
        cen * (lax.rsqrt(var + 1e-5) * g_ref[...])[None, :]
        + be_ref[...][None, :] + x_ref[...]
    )


def kernel(x, edge_index, bit_sum, W, bias, bn_gamma, bn_beta):
    row = edge_index[0]
    col = edge_index[1]
    degf = _deg_call(col)
    g, dis = pl.pallas_call(
        _lin_body,
        out_shape=[
            jax.ShapeDtypeStruct((_N, _D), jnp.float32),
            jax.ShapeDtypeStruct((_NPAD,), jnp.float32),
        ],
    )(x, W, degf.reshape(_NC, _NPAD))
    sflat = _msg_call(row, col, g)
    out = pl.pallas_call(
        _bn_body,
        out_shape=jax.ShapeDtypeStruct((_N, _D), jnp.float32),
    )(sflat, dis, bias, bn_gamma, bn_beta, x)
    return (out, jnp.asarray(0, dtype=jnp.int32))


# speedup vs baseline: 37.6782x; 2.1320x over previous
import functools

import jax
import jax.numpy as jnp
from jax import lax
from jax.experimental import pallas as pl
from jax.experimental.pallas import tpu as pltpu
from jax.experimental.pallas import tpu_sc as plsc

_N = 10000
_E = 320000
_D = 128
_NC = 2
_NS = 16
_NW = _NC * _NS
_NPAD = 10240
_RPT = _NPAD // _NS
_CH = 128
_CPT = 80
_EPAD = _NW * _CPT * _CH

_MESH = dict(core_axis_name="c", subcore_axis_name="s")


@functools.partial(
    pl.kernel,
    out_type=jax.ShapeDtypeStruct((_NC * _NPAD,), jnp.float32),
    mesh=plsc.VectorSubcoreMesh(**_MESH),
    scratch_types=[
        pltpu.VMEM((_CPT, _CH), jnp.int32),
        pltpu.VMEM((_CH,), jnp.float32),
        pltpu.VMEM((_RPT,), jnp.float32),
        pltpu.VMEM_SHARED((_NPAD,), jnp.float32),
        pltpu.SemaphoreType.DMA,
        pltpu.SemaphoreType.DMA,
    ],
)
def _deg_call(col_hbm, deg_hbm, coli2, ones_c, zbuf, deg_sh, psem, ssem):
    cid = lax.axis_index("c")
    sid = lax.axis_index("s")
    w = cid * _NS + sid
    pre = pltpu.async_copy(col_hbm.at[pl.ds(w * _CPT, _CPT)], coli2, psem)
    zeros16 = jnp.zeros((16,), jnp.float32)
    ones16 = jnp.ones((16,), jnp.float32)
    for j in range(_CH // 16):
        ones_c[pl.ds(j * 16, 16)] = ones16
    for j in range(_RPT // 16):
        zbuf[pl.ds(j * 16, 16)] = zeros16
    pltpu.sync_copy(zbuf, deg_sh.at[pl.ds(sid * _RPT, _RPT)])
    pre.wait()
    plsc.subcore_barrier()

    def fire(j, carry):
        pltpu.async_copy(ones_c, deg_sh.at[coli2.at[j]], ssem, add=True)
        return carry

    lax.fori_loop(0, _CPT, fire, 0)

    def drain(j, carry):
        pltpu.make_async_copy(ones_c, deg_sh.at[coli2.at[j]], ssem).wait()
        return carry

    lax.fori_loop(0, _CPT, drain, 0)
    plsc.subcore_barrier()
    pltpu.sync_copy(
        deg_sh.at[pl.ds(sid * _RPT, _RPT)],
        deg_hbm.at[pl.ds(cid * _NPAD + sid * _RPT, _RPT)],
    )


@functools.partial(
    pl.kernel,
    out_type=jax.ShapeDtypeStruct((_NC * _NPAD, _D), jnp.float32),
    mesh=plsc.VectorSubcoreMesh(**_MESH),
    scratch_types=[
        pltpu.VMEM((_CPT, _CH), jnp.int32),
        pltpu.VMEM((_CH,), jnp.int32),
        pltpu.VMEM((_CH,), jnp.int32),
        pltpu.VMEM((_CH, _D), jnp.float32),
        pltpu.VMEM((_CH, _D), jnp.float32),
        pltpu.VMEM((16, _D), jnp.float32),
        pltpu.VMEM_SHARED((_NPAD, _D), jnp.float32),
        pltpu.SemaphoreType.DMA,
        pltpu.SemaphoreType.DMA,
        pltpu.SemaphoreType.DMA,
        pltpu.SemaphoreType.DMA,
    ],
)
def _msg_call(row_hbm, col_hbm, g_hbm, out_hbm,
              rowi2, colia, colib, bufa, bufb, zb, acc_sh,
              psem, zsem, sema, semb):
    cid = lax.axis_index("c")
    sid = lax.axis_index("s")
    w = cid * _NS + sid
    ebase = w * (_CPT * _CH)
    prer = pltpu.async_copy(row_hbm.at[pl.ds(w * _CPT, _CPT)], rowi2, psem)
    zeros16 = jnp.zeros((16,), jnp.float32)
    for i in range(16):
        for j in range(_D // 16):
            zb[i, pl.ds(j * 16, 16)] = zeros16

    def zfire(t, carry):
        pltpu.async_copy(zb, acc_sh.at[pl.ds(sid * _RPT + t * 16, 16)], zsem)
        return carry

    lax.fori_loop(0, _RPT // 16, zfire, 0)

    def zdrain(t, carry):
        pltpu.make_async_copy(
            zb, acc_sh.at[pl.ds(sid * _RPT + t * 16, 16)], zsem).wait()
        return carry

    lax.fori_loop(0, _RPT // 16, zdrain, 0)
    prer.wait()
    plsc.subcore_barrier()

    def cref(j):
        return col_hbm.at[pl.ds(ebase + j * _CH, _CH)]

    def start(j, cbuf, gbuf, sem):
        pltpu.async_copy(cref(j), cbuf, sem)
        pltpu.async_copy(g_hbm.at[rowi2.at[j]], gbuf, sem)

    def finish(j, cbuf, gbuf, sem):
        pltpu.make_async_copy(cref(j), cbuf, sem).wait()
        pltpu.make_async_copy(g_hbm.at[rowi2.at[j]], gbuf, sem).wait()
        pltpu.sync_copy(gbuf, acc_sh.at[cbuf], add=True)

    start(0, colia, bufa, sema)

    def body(k, carry):
        j = 2 * k
        start(j + 1, colib, bufb, semb)
        finish(j, colia, bufa, sema)
        start(j + 2, colia, bufa, sema)
        finish(j + 1, colib, bufb, semb)
        return carry

    lax.fori_loop(0, (_CPT - 2) // 2, body, 0)
    start(_CPT - 1, colib, bufb, semb)
    finish(_CPT - 2, colia, bufa, sema)
    finish(_CPT - 1, colib, bufb, semb)
    plsc.subcore_barrier()
    pltpu.sync_copy(
        acc_sh.at[pl.ds(sid * _RPT, _RPT)],
        out_hbm.at[pl.ds(cid * _NPAD + sid * _RPT, _RPT)],
    )


def _lin_body(x_ref, w_ref, degp_ref, g_ref, dis_ref):
    deg = degp_ref[0, :] + degp_ref[1, :]
    dis = jnp.where(deg > 0.0, lax.rsqrt(deg), 0.0)
    dis_ref[...] = dis
    h = lax.dot_general(
        x_ref[...], w_ref[...], (((1,), (1,)), ((), ())),
        preferred_element_type=jnp.float32,
    )
    g_ref[...] = h * dis[:_N][:, None]


def _bn_body(s_ref, dis_ref, b_ref, g_ref, be_ref, x_ref, o_ref):
    s = s_ref[pl.ds(0, _N), :] + s_ref[pl.ds(_NPAD, _N), :]
    pre = s * dis_ref[pl.ds(0, _N)][:, None] + b_ref[...][None, :]
    mean = jnp.mean(pre, axis=0)
    cen = pre - mean[None, :]
    var = jnp.mean(cen * cen, axis=0)
    o_ref[...] = (
        cen * (lax.rsqrt(var + 1e-5) * g_ref[...])[None, :]
        + be_ref[...][None, :] + x_ref[...]
    )


def kernel(x, edge_index, bit_sum, W, bias, bn_gamma, bn_beta):
    npad = _EPAD - _E
    ar = jnp.arange(npad, dtype=jnp.int32)
    row2d = jnp.concatenate([edge_index[0], ar % _N]).reshape(-1, _CH)
    col2d = jnp.concatenate(
        [edge_index[1], _N + ar % (_NPAD - _N)]).reshape(-1, _CH)
    degf = _deg_call(col2d)
    g, dis = pl.pallas_call(
        _lin_body,
        out_shape=[
            jax.ShapeDtypeStruct((_N, _D), jnp.float32),
            jax.ShapeDtypeStruct((_NPAD,), jnp.float32),
        ],
    )(x, W, degf.reshape(_NC, _NPAD))
    sflat = _msg_call(row2d, col2d.reshape(-1), g)
    out = pl.pallas_call(
        _bn_body,
        out_shape=jax.ShapeDtypeStruct((_N, _D), jnp.float32),
    )(sflat, dis, bias, bn_gamma, bn_beta, x)
    return (out, jnp.asarray(0, dtype=jnp.int32))
